# Initial kernel scaffold; baseline (speedup 1.0000x reference)
import jax
import jax.numpy as jnp
from jax.experimental import pallas as pl


def kernel(boxes, scores):
    raise NotImplementedError("write your pallas kernel here")


# speedup vs baseline: 72.7352x; 72.7352x over previous
import functools

import jax
import jax.numpy as jnp
from jax.experimental import pallas as pl
from jax.experimental.pallas import tpu as pltpu

_N = 20000
_K = 4096
_B = 128
_NB = _K // _B
_POST = 100
_THR = 0.7
_SCORE_T = 0.05
_IMG = 1024.0


def _nms_body(a_ref, keep_ref, m_ref, v_ref):
    sub = jax.lax.broadcasted_iota(jnp.int32, (_B, _B), 0)
    lan = jax.lax.broadcasted_iota(jnp.int32, (_B, _B), 1)
    eye = (sub == lan).astype(jnp.float32)
    lane_row = jax.lax.broadcasted_iota(jnp.int32, (1, _B), 1)

    def tcol(row):
        return jnp.sum(eye * row, axis=1, keepdims=True)

    def block_j(j, carry):
        Aj = a_ref[pl.ds(j, 1)].reshape(8, _B)
        x0c, y0c, x1c, y1c = Aj[0:1], Aj[1:2], Aj[2:3], Aj[3:4]
        validj = Aj[4:5]
        x0r, y0r, x1r, y1r = tcol(x0c), tcol(y0c), tcol(x1c), tcol(y1c)
        area_r = (x1r - x0r) * (y1r - y0r)

        def iou_mask(px0, py0, px1, py1, parea):
            w = jnp.maximum(jnp.minimum(x1r, px1) - jnp.maximum(x0r, px0), 0.0)
            h = jnp.maximum(jnp.minimum(y1r, py1) - jnp.maximum(y0r, py0), 0.0)
            inter = w * h
            union = jnp.maximum(area_r + parea - inter, 1e-8)
            return inter > _THR * union

        def tile_p(p, supc):
            Ap = a_ref[pl.ds(p, 1)].reshape(8, _B)
            parea = (Ap[2:3] - Ap[0:1]) * (Ap[3:4] - Ap[1:2])
            m = iou_mask(Ap[0:1], Ap[1:2], Ap[2:3], Ap[3:4], parea)
            keep_p = keep_ref[pl.ds(p, 1)].reshape(1, _B)
            mf = m.astype(jnp.float32) * keep_p
            return jnp.maximum(supc, jnp.max(mf, axis=1, keepdims=True))

        supc = jax.lax.fori_loop(
            0, j, tile_p, jnp.zeros((_B, 1), jnp.float32))

        area_c = (x1c - x0c) * (y1c - y0c)
        Mf = (iou_mask(x0c, y0c, x1c, y1c, area_c) & (lan < sub)).astype(
            jnp.float32)
        sup_row = jnp.sum(eye * supc, axis=0, keepdims=True)
        v0_row = jnp.where(sup_row > 0.0, 0.0, validj)

        def seq_resolve():
            m_ref[...] = Mf
            v_ref[...] = tcol(v0_row)

            def inner(i, kv):
                row = m_ref[pl.ds(i, 1), :]
                s = jnp.max(row * kv, axis=1, keepdims=True)
                vi = v_ref[pl.ds(i, 1), :]
                newv = jnp.where(s > 0.0, 0.0, vi)
                return jnp.where(lane_row == i, newv, kv)

            return jax.lax.fori_loop(
                0, _B, inner, jnp.zeros((1, _B), jnp.float32))

        kv = jax.lax.cond(jnp.max(Mf) > 0.0, seq_resolve, lambda: v0_row)
        keep_ref[pl.ds(j, 1)] = kv[None]
        return carry

    jax.lax.fori_loop(0, _NB, block_j, 0)


def kernel(boxes, scores):
    boxes = jnp.clip(boxes, 0.0, _IMG)
    scores = jnp.where(scores >= _SCORE_T, scores, 0.0)
    top_scores, idx = jax.lax.top_k(scores, _K)
    top_boxes = jnp.take(boxes, idx, axis=0)

    coords = top_boxes.T.reshape(4, _NB, _B).transpose(1, 0, 2)
    valid = (top_scores > 0.0).astype(jnp.float32).reshape(_NB, 1, _B)
    pad = jnp.zeros((_NB, 3, _B), jnp.float32)
    A = jnp.concatenate([coords, valid, pad], axis=1)

    keep = pl.pallas_call(
        _nms_body,
        out_shape=jax.ShapeDtypeStruct((_NB, 1, _B), jnp.float32),
        scratch_shapes=[
            pltpu.VMEM((_B, _B), jnp.float32),
            pltpu.VMEM((_B, 1), jnp.float32),
        ],
    )(A)
    keep = keep.reshape(_K)

    kept_scores = jnp.where(keep > 0.0, top_scores, -1.0)
    final_scores, fidx = jax.lax.top_k(kept_scores, _POST)
    final_boxes = jnp.take(top_boxes, fidx, axis=0)
    final_scores = jnp.maximum(final_scores, 0.0)
    return jnp.concatenate([final_boxes, final_scores[:, None]], axis=-1)


# speedup vs baseline: 73.2396x; 1.0069x over previous
import jax
import jax.numpy as jnp
from jax.experimental import pallas as pl
from jax.experimental.pallas import tpu as pltpu

_N = 20000
_K = 4096
_B = 128
_NB = _K // _B
_POST = 100
_THR = 0.7
_SCORE_T = 0.05
_IMG = 1024.0


def _nms_body(a_ref, w_ref, keep_ref, sup_ref, m_ref, v_ref):
    sub = jax.lax.broadcasted_iota(jnp.int32, (_B, _B), 0)
    lan = jax.lax.broadcasted_iota(jnp.int32, (_B, _B), 1)
    eye = (sub == lan).astype(jnp.float32)
    lane_row = jax.lax.broadcasted_iota(jnp.int32, (1, _B), 1)
    pos_wide = jax.lax.broadcasted_iota(jnp.int32, (1, _K), 1)

    x0w = w_ref[0:1, :]
    y0w = w_ref[1:2, :]
    x1w = w_ref[2:3, :]
    y1w = w_ref[3:4, :]
    areaw = w_ref[5:6, :]

    sup_ref[...] = jnp.zeros((_NB, 1, _B), jnp.float32)

    def tcol(row):
        return jnp.sum(eye * row, axis=1, keepdims=True)

    def block_j(j, carry):
        Aj = a_ref[pl.ds(j, 1)].reshape(8, _B)
        x0c, y0c, x1c, y1c = Aj[0:1], Aj[1:2], Aj[2:3], Aj[3:4]
        validj, areac = Aj[4:5], Aj[5:6]
        x0r, y0r, x1r, y1r = tcol(x0c), tcol(y0c), tcol(x1c), tcol(y1c)
        area_r = tcol(areac)

        wj = jnp.maximum(jnp.minimum(x1r, x1c) - jnp.maximum(x0r, x0c), 0.0)
        hj = jnp.maximum(jnp.minimum(y1r, y1c) - jnp.maximum(y0r, y0c), 0.0)
        interj = wj * hj
        unionj = jnp.maximum(area_r + areac - interj, 1e-8)
        Mf = ((interj > _THR * unionj) & (lan < sub)).astype(jnp.float32)

        supj = sup_ref[pl.ds(j, 1)].reshape(1, _B)
        v0_row = jnp.where(supj > 0.0, 0.0, validj)

        def seq_resolve():
            m_ref[...] = Mf
            v_ref[...] = tcol(v0_row)

            def inner(i, kv):
                row = m_ref[pl.ds(i, 1), :]
                s = jnp.max(row * kv, axis=1, keepdims=True)
                vi = v_ref[pl.ds(i, 1), :]
                newv = jnp.where(s > 0.0, 0.0, vi)
                return jnp.where(lane_row == i, newv, kv)

            return jax.lax.fori_loop(
                0, _B, inner, jnp.zeros((1, _B), jnp.float32))

        kv = jax.lax.cond(jnp.max(Mf) > 0.0, seq_resolve, lambda: v0_row)
        keep_ref[pl.ds(j, 1)] = kv[None]

        kcol = tcol(kv)
        ww = jnp.maximum(jnp.minimum(x1r, x1w) - jnp.maximum(x0r, x0w), 0.0)
        hw = jnp.maximum(jnp.minimum(y1r, y1w) - jnp.maximum(y0r, y0w), 0.0)
        interw = ww * hw
        unionw = jnp.maximum(area_r + areaw - interw, 1e-8)
        mw = (interw > _THR * unionw).astype(jnp.float32) * kcol
        upd = jnp.max(mw, axis=0, keepdims=True)
        upd = jnp.where(pos_wide >= (j + 1) * _B, upd, 0.0)
        for k in range(_NB):
            chunk = upd[:, k * _B:(k + 1) * _B][None]
            sup_ref[k:k + 1] = jnp.maximum(sup_ref[k:k + 1], chunk)
        return carry

    jax.lax.fori_loop(0, _NB, block_j, 0)


def kernel(boxes, scores):
    boxes = jnp.clip(boxes, 0.0, _IMG)
    scores = jnp.where(scores >= _SCORE_T, scores, 0.0)
    top_scores, idx = jax.lax.top_k(scores, _K)
    top_boxes = jnp.take(boxes, idx, axis=0)

    coords_w = top_boxes.T
    valid_w = (top_scores > 0.0).astype(jnp.float32)[None]
    area_w = ((coords_w[2:3] - coords_w[0:1])
              * (coords_w[3:4] - coords_w[1:2]))
    W = jnp.concatenate(
        [coords_w, valid_w, area_w, jnp.zeros((2, _K), jnp.float32)], axis=0)
    A = W.reshape(8, _NB, _B).transpose(1, 0, 2)

    keep = pl.pallas_call(
        _nms_body,
        out_shape=jax.ShapeDtypeStruct((_NB, 1, _B), jnp.float32),
        scratch_shapes=[
            pltpu.VMEM((_NB, 1, _B), jnp.float32),
            pltpu.VMEM((_B, _B), jnp.float32),
            pltpu.VMEM((_B, 1), jnp.float32),
        ],
    )(A, W)
    keep = keep.reshape(_K)

    kept_scores = jnp.where(keep > 0.0, top_scores, -1.0)
    final_scores, fidx = jax.lax.top_k(kept_scores, _POST)
    final_boxes = jnp.take(top_boxes, fidx, axis=0)
    final_scores = jnp.maximum(final_scores, 0.0)
    return jnp.concatenate([final_boxes, final_scores[:, None]], axis=-1)


# speedup vs baseline: 264.1658x; 3.6069x over previous
import jax
import jax.numpy as jnp
from jax.experimental import pallas as pl
from jax.experimental.pallas import tpu as pltpu

_N = 20000
_K = 4096
_B = 128
_NB = _K // _B
_POST = 100
_THR = 0.7
_SCORE_T = 0.05
_IMG = 1024.0


def _nms_body(a_ref, w_ref, keep_ref, sup_ref):
    sub = jax.lax.broadcasted_iota(jnp.int32, (_B, _B), 0)
    lan = jax.lax.broadcasted_iota(jnp.int32, (_B, _B), 1)
    eye = (sub == lan).astype(jnp.float32)
    pos_wide = jax.lax.broadcasted_iota(jnp.int32, (1, _K), 1)

    x0w = w_ref[0:1, :]
    y0w = w_ref[1:2, :]
    x1w = w_ref[2:3, :]
    y1w = w_ref[3:4, :]
    areaw = w_ref[5:6, :]

    sup_ref[...] = jnp.zeros((_NB, 1, _B), jnp.float32)

    def tcol(row):
        return jnp.sum(eye * row, axis=1, keepdims=True)

    def block_j(j, carry):
        Aj = a_ref[pl.ds(j, 1)].reshape(8, _B)
        x0c, y0c, x1c, y1c = Aj[0:1], Aj[1:2], Aj[2:3], Aj[3:4]
        validj, areac = Aj[4:5], Aj[5:6]
        x0r, y0r, x1r, y1r = tcol(x0c), tcol(y0c), tcol(x1c), tcol(y1c)
        area_r = tcol(areac)

        wj = jnp.maximum(jnp.minimum(x1r, x1c) - jnp.maximum(x0r, x0c), 0.0)
        hj = jnp.maximum(jnp.minimum(y1r, y1c) - jnp.maximum(y0r, y0c), 0.0)
        interj = wj * hj
        unionj = jnp.maximum(area_r + areac - interj, 1e-8)
        Mf = ((interj > _THR * unionj) & (lan < sub)).astype(jnp.float32)

        supj = sup_ref[pl.ds(j, 1)].reshape(1, _B)
        v0_row = jnp.where(supj > 0.0, 0.0, validj)
        v0c = tcol(v0_row)

        def fix_body(state):
            kc, _ = state
            supv = jnp.dot(Mf, kc, preferred_element_type=jnp.float32)
            knew = jnp.where(supv > 0.0, 0.0, v0c)
            changed = jnp.max(jnp.abs(knew - kc))
            return knew, changed

        kcol, _ = jax.lax.while_loop(
            lambda s: s[1] > 0.0, fix_body, (v0c, jnp.float32(1.0)))

        kv = jnp.sum(eye * kcol, axis=0, keepdims=True)
        keep_ref[pl.ds(j, 1)] = kv[None]

        ww = jnp.maximum(jnp.minimum(x1r, x1w) - jnp.maximum(x0r, x0w), 0.0)
        hw = jnp.maximum(jnp.minimum(y1r, y1w) - jnp.maximum(y0r, y0w), 0.0)
        interw = ww * hw
        unionw = jnp.maximum(area_r + areaw - interw, 1e-8)
        mw = (interw > _THR * unionw).astype(jnp.float32) * kcol
        upd = jnp.max(mw, axis=0, keepdims=True)
        upd = jnp.where(pos_wide >= (j + 1) * _B, upd, 0.0)
        for k in range(_NB):
            chunk = upd[:, k * _B:(k + 1) * _B][None]
            sup_ref[k:k + 1] = jnp.maximum(sup_ref[k:k + 1], chunk)
        return carry

    jax.lax.fori_loop(0, _NB, block_j, 0)


def kernel(boxes, scores):
    boxes = jnp.clip(boxes, 0.0, _IMG)
    scores = jnp.where(scores >= _SCORE_T, scores, 0.0)
    top_scores, idx = jax.lax.top_k(scores, _K)
    top_boxes = jnp.take(boxes, idx, axis=0)

    coords_w = top_boxes.T
    valid_w = (top_scores > 0.0).astype(jnp.float32)[None]
    area_w = ((coords_w[2:3] - coords_w[0:1])
              * (coords_w[3:4] - coords_w[1:2]))
    W = jnp.concatenate(
        [coords_w, valid_w, area_w, jnp.zeros((2, _K), jnp.float32)], axis=0)
    A = W.reshape(8, _NB, _B).transpose(1, 0, 2)

    keep = pl.pallas_call(
        _nms_body,
        out_shape=jax.ShapeDtypeStruct((_NB, 1, _B), jnp.float32),
        scratch_shapes=[
            pltpu.VMEM((_NB, 1, _B), jnp.float32),
        ],
    )(A, W)
    keep = keep.reshape(_K)

    kept_scores = jnp.where(keep > 0.0, top_scores, -1.0)
    final_scores, fidx = jax.lax.top_k(kept_scores, _POST)
    final_boxes = jnp.take(top_boxes, fidx, axis=0)
    final_scores = jnp.maximum(final_scores, 0.0)
    return jnp.concatenate([final_boxes, final_scores[:, None]], axis=-1)


# speedup vs baseline: 345.2809x; 1.3071x over previous
import jax
import jax.numpy as jnp
from jax.experimental import pallas as pl
from jax.experimental.pallas import tpu as pltpu

_N = 20000
_K = 4096
_B = 128
_NB = _K // _B
_POST = 100
_THR = 0.7
_SCORE_T = 0.05
_IMG = 1024.0


def _nms_body(a_ref, w_ref, keep_ref, sup_ref):
    sub = jax.lax.broadcasted_iota(jnp.int32, (_B, _B), 0)
    lan = jax.lax.broadcasted_iota(jnp.int32, (_B, _B), 1)
    eye = (sub == lan).astype(jnp.float32)

    x0w = w_ref[0:1, :]
    y0w = w_ref[1:2, :]
    x1w = w_ref[2:3, :]
    y1w = w_ref[3:4, :]
    areaw = w_ref[5:6, :]

    sup_ref[...] = jnp.zeros((_NB, 1, _B), jnp.float32)

    def tcol(row):
        return jnp.sum(eye * row, axis=1, keepdims=True)

    def block_j(j, carry):
        Aj = a_ref[pl.ds(j, 1)].reshape(8, _B)
        x0c, y0c, x1c, y1c = Aj[0:1], Aj[1:2], Aj[2:3], Aj[3:4]
        validj, areac = Aj[4:5], Aj[5:6]
        x0r, y0r, x1r, y1r = tcol(x0c), tcol(y0c), tcol(x1c), tcol(y1c)
        area_r = tcol(areac)

        wj = jnp.maximum(jnp.minimum(x1r, x1c) - jnp.maximum(x0r, x0c), 0.0)
        hj = jnp.maximum(jnp.minimum(y1r, y1c) - jnp.maximum(y0r, y0c), 0.0)
        interj = wj * hj
        unionj = jnp.maximum(area_r + areac - interj, 1e-8)
        Mf = ((interj > _THR * unionj) & (lan < sub)).astype(jnp.float32)

        supj = sup_ref[pl.ds(j, 1)].reshape(1, _B)
        v0_row = jnp.where(supj > 0.0, 0.0, validj)
        v0c = tcol(v0_row)

        def fix_body(state):
            kc, _ = state
            supv = jnp.dot(Mf, kc, preferred_element_type=jnp.float32)
            knew = jnp.where(supv > 0.0, 0.0, v0c)
            changed = jnp.max(jnp.abs(knew - kc))
            return knew, changed

        kcol, _ = jax.lax.while_loop(
            lambda s: s[1] > 0.0, fix_body, (v0c, jnp.float32(1.0)))

        kv = jnp.sum(eye * kcol, axis=0, keepdims=True)
        keep_ref[pl.ds(j, 1)] = kv[None]

        kq = _K // 4
        nbq = kq // _B
        for q in range(4):
            c0 = q * kq

            @pl.when(j < q * nbq + nbq - 1)
            def _():
                x0q = w_ref[0:1, c0:c0 + kq]
                y0q = w_ref[1:2, c0:c0 + kq]
                x1q = w_ref[2:3, c0:c0 + kq]
                y1q = w_ref[3:4, c0:c0 + kq]
                areaq = w_ref[5:6, c0:c0 + kq]
                ww = jnp.maximum(
                    jnp.minimum(x1r, x1q) - jnp.maximum(x0r, x0q), 0.0)
                hw = jnp.maximum(
                    jnp.minimum(y1r, y1q) - jnp.maximum(y0r, y0q), 0.0)
                interw = ww * hw
                unionw = jnp.maximum(area_r + areaq - interw, 1e-8)
                mw = (interw > _THR * unionw).astype(jnp.float32) * kcol
                upd = jnp.max(mw, axis=0, keepdims=True)
                posq = c0 + jax.lax.broadcasted_iota(jnp.int32, (1, kq), 1)
                upd = jnp.where(posq >= (j + 1) * _B, upd, 0.0)
                for k in range(nbq):
                    kk = q * nbq + k
                    chunk = upd[:, k * _B:(k + 1) * _B][None]
                    sup_ref[kk:kk + 1] = jnp.maximum(
                        sup_ref[kk:kk + 1], chunk)
        return carry

    jax.lax.fori_loop(0, _NB, block_j, 0)


def kernel(boxes, scores):
    boxes = jnp.clip(boxes, 0.0, _IMG)
    scores = jnp.where(scores >= _SCORE_T, scores, 0.0)
    top_scores, idx = jax.lax.top_k(scores, _K)
    top_boxes = jnp.take(boxes, idx, axis=0)

    coords_w = top_boxes.T
    valid_w = (top_scores > 0.0).astype(jnp.float32)[None]
    area_w = ((coords_w[2:3] - coords_w[0:1])
              * (coords_w[3:4] - coords_w[1:2]))
    W = jnp.concatenate(
        [coords_w, valid_w, area_w, jnp.zeros((2, _K), jnp.float32)], axis=0)
    A = W.reshape(8, _NB, _B).transpose(1, 0, 2)

    keep = pl.pallas_call(
        _nms_body,
        out_shape=jax.ShapeDtypeStruct((_NB, 1, _B), jnp.float32),
        scratch_shapes=[
            pltpu.VMEM((_NB, 1, _B), jnp.float32),
        ],
    )(A, W)
    keep = keep.reshape(_K)

    kept_scores = jnp.where(keep > 0.0, top_scores, -1.0)
    final_scores, fidx = jax.lax.top_k(kept_scores, _POST)
    final_boxes = jnp.take(top_boxes, fidx, axis=0)
    final_scores = jnp.maximum(final_scores, 0.0)
    return jnp.concatenate([final_boxes, final_scores[:, None]], axis=-1)
